# baseline (speedup 1.0000x reference)
import functools

import jax
import jax.numpy as jnp
from jax import lax
from jax.experimental import pallas as pl
from jax.experimental.pallas import tpu as pltpu
from jax.experimental.pallas import tpu_sc as plsc

_B, _D1, _D2 = 64, 8192, 128
_DTC = 3584
_DSC = _D1 - _DTC


_UNROLL = 8


def _tc_body(x_ref, oval_ref, oidx_ref):
    d1 = x_ref.shape[1]
    rows_per_iter = 8 * _UNROLL
    n_iter = d1 // rows_per_iter

    sub_iota = lax.broadcasted_iota(jnp.int32, (8, _D2), 0)

    def step(i, carry):
        rm, ri = carry
        v64 = x_ref[0, pl.ds(i * rows_per_iter, rows_per_iter), :]
        vv = v64.reshape(_UNROLL, 8, _D2)
        base = i * rows_per_iter
        for k in range(_UNROLL):
            v = vv[k]
            lt = v < rm
            ri = jnp.where(lt, sub_iota + (base + k * 8), ri)
            rm = jnp.where(lt, v, rm)
        return rm, ri

    rm0 = jnp.full((8, _D2), jnp.inf, dtype=jnp.float32)
    ri0 = jnp.zeros((8, _D2), dtype=jnp.int32)
    rm, ri = lax.fori_loop(0, n_iter, step, (rm0, ri0))

    m = jnp.min(rm, axis=0)
    idx = jnp.min(jnp.where(rm == m[None, :], ri, 2**30), axis=0)
    oval_ref[0, 0] = m
    oidx_ref[0, 0] = idx


def _tc_argmin(x):
    val3, idx3 = pl.pallas_call(
        _tc_body,
        grid=(_B,),
        in_specs=[pl.BlockSpec((1, _DTC, _D2), lambda i: (i, 0, 0))],
        out_specs=[
            pl.BlockSpec((1, 1, _D2), lambda i: (i, 0, 0)),
            pl.BlockSpec((1, 1, _D2), lambda i: (i, 0, 0)),
        ],
        out_shape=[
            jax.ShapeDtypeStruct((_B, 1, _D2), jnp.float32),
            jax.ShapeDtypeStruct((_B, 1, _D2), jnp.int32),
        ],
    )(x)
    return val3.reshape(_B, _D2), idx3.reshape(_B, _D2)


_NC, _NS, _L = 2, 16, 16
_NW = _NC * _NS
_BPW = _B // _NW
_C = 256
_NCHUNK = _DSC // _C
_G = _D2 // _L


def _sc_body(x_hbm, oidx_hbm, oval_hbm, buf0, buf1, outi, outf, sem0, sem1):
    wid = lax.axis_index("s") * _NC + lax.axis_index("c")
    bufs = (buf0, buf1)
    sems = (sem0, sem1)

    def chunk_src(b, g):
        return x_hbm.at[b, pl.ds(_DTC + g * _C, _C), :]

    for bi in range(_BPW):
        b = wid * _BPW + bi
        pltpu.make_async_copy(chunk_src(b, 0), buf0, sem0).start()
        pltpu.make_async_copy(chunk_src(b, 1), buf1, sem1).start()

        def row_body(d, carry, buf, g):
            vals = carry[:_G]
            idxs = carry[_G:]
            new_vals, new_idxs = [], []
            dvec = jnp.full((_L,), _DTC + g * _C + d, dtype=jnp.int32)
            for l in range(_G):
                v = buf[d, pl.ds(l * _L, _L)]
                lt = v < vals[l]
                new_idxs.append(jnp.where(lt, dvec, idxs[l]))
                new_vals.append(jnp.where(lt, v, vals[l]))
            return tuple(new_vals) + tuple(new_idxs)

        def pair_body(p, carry):
            for j in range(2):
                g = p * 2 + j
                buf, sem = bufs[j], sems[j]
                pltpu.make_async_copy(chunk_src(b, g), buf, sem).wait()
                carry = lax.fori_loop(
                    0, _C, functools.partial(row_body, buf=buf, g=g), carry,
                    unroll=4)

                @pl.when(g + 2 < _NCHUNK)
                def _():
                    pltpu.make_async_copy(chunk_src(b, g + 2), buf, sem).start()
            return carry

        init = tuple(jnp.full((_L,), jnp.inf, dtype=jnp.float32)
                     for _ in range(_G))
        init += tuple(jnp.zeros((_L,), dtype=jnp.int32) for _ in range(_G))
        carry = lax.fori_loop(0, _NCHUNK // 2, pair_body, init)

        for l in range(_G):
            outf[bi, pl.ds(l * _L, _L)] = carry[l]
            outi[bi, pl.ds(l * _L, _L)] = carry[_G + l]

    pltpu.sync_copy(outi, oidx_hbm.at[pl.ds(wid * _BPW, _BPW), :])
    pltpu.sync_copy(outf, oval_hbm.at[pl.ds(wid * _BPW, _BPW), :])


def _sc_argmin(x):
    mesh = plsc.VectorSubcoreMesh(core_axis_name="c", subcore_axis_name="s")
    f = pl.kernel(
        _sc_body,
        out_type=(
            jax.ShapeDtypeStruct((_B, _D2), jnp.int32),
            jax.ShapeDtypeStruct((_B, _D2), jnp.float32),
        ),
        mesh=mesh,
        scratch_types=[
            pltpu.VMEM((_C, _D2), jnp.float32),
            pltpu.VMEM((_C, _D2), jnp.float32),
            pltpu.VMEM((_BPW, _D2), jnp.int32),
            pltpu.VMEM((_BPW, _D2), jnp.float32),
            pltpu.SemaphoreType.DMA,
            pltpu.SemaphoreType.DMA,
        ],
    )
    return f(x)


def _merge_body(tv_ref, ti_ref, sv_ref, si_ref, o_ref):
    sc_wins = sv_ref[...] < tv_ref[...]
    o_ref[...] = jnp.where(sc_wins, si_ref[...], ti_ref[...])


def _merge(tval, tidx, sval, sidx):
    return pl.pallas_call(
        _merge_body,
        out_shape=jax.ShapeDtypeStruct((_B, _D2), jnp.int32),
    )(tval, tidx, sval, sidx)


def kernel(x):
    sidx, sval = _sc_argmin(x)
    tval, tidx = _tc_argmin(x)
    return _merge(tval, tidx, sval, sidx)


# speedup vs baseline: 1.0906x; 1.0906x over previous
import functools

import jax
import jax.numpy as jnp
from jax import lax
from jax.experimental import pallas as pl
from jax.experimental.pallas import tpu as pltpu
from jax.experimental.pallas import tpu_sc as plsc

_B, _D1, _D2 = 64, 8192, 128
_DTC = 4096
_DSC = _D1 - _DTC


_UNROLL = 8


def _tc_body(x_ref, oval_ref, oidx_ref):
    d1 = x_ref.shape[1]
    rows_per_iter = 8 * _UNROLL
    n_iter = d1 // rows_per_iter

    sub_iota = lax.broadcasted_iota(jnp.int32, (8, _D2), 0)

    def step(i, carry):
        rm, ri = carry
        v64 = x_ref[0, pl.ds(i * rows_per_iter, rows_per_iter), :]
        vv = v64.reshape(_UNROLL, 8, _D2)
        base = i * rows_per_iter
        for k in range(_UNROLL):
            v = vv[k]
            lt = v < rm
            ri = jnp.where(lt, sub_iota + (base + k * 8), ri)
            rm = jnp.where(lt, v, rm)
        return rm, ri

    rm0 = jnp.full((8, _D2), jnp.inf, dtype=jnp.float32)
    ri0 = jnp.zeros((8, _D2), dtype=jnp.int32)
    rm, ri = lax.fori_loop(0, n_iter, step, (rm0, ri0))

    m = jnp.min(rm, axis=0)
    idx = jnp.min(jnp.where(rm == m[None, :], ri, 2**30), axis=0)
    oval_ref[0, 0] = m
    oidx_ref[0, 0] = idx


def _tc_argmin(x):
    val3, idx3 = pl.pallas_call(
        _tc_body,
        grid=(_B,),
        in_specs=[pl.BlockSpec((1, _DTC, _D2), lambda i: (i, 0, 0))],
        out_specs=[
            pl.BlockSpec((1, 1, _D2), lambda i: (i, 0, 0)),
            pl.BlockSpec((1, 1, _D2), lambda i: (i, 0, 0)),
        ],
        out_shape=[
            jax.ShapeDtypeStruct((_B, 1, _D2), jnp.float32),
            jax.ShapeDtypeStruct((_B, 1, _D2), jnp.int32),
        ],
    )(x)
    return val3.reshape(_B, _D2), idx3.reshape(_B, _D2)


_NC, _NS, _L = 2, 16, 16
_NW = _NC * _NS
_BPW = _B // _NW
_C = 256
_NCHUNK = _DSC // _C
_G = _D2 // _L


def _sc_body(x_hbm, oidx_hbm, oval_hbm, buf0, buf1, outi, outf, sem0, sem1):
    wid = lax.axis_index("s") * _NC + lax.axis_index("c")
    bufs = (buf0, buf1)
    sems = (sem0, sem1)

    def chunk_src(b, g):
        return x_hbm.at[b, pl.ds(_DTC + g * _C, _C), :]

    for bi in range(_BPW):
        b = wid * _BPW + bi
        pltpu.make_async_copy(chunk_src(b, 0), buf0, sem0).start()
        pltpu.make_async_copy(chunk_src(b, 1), buf1, sem1).start()

        def row_body(d, carry, buf, g):
            vals = carry[:_G]
            idxs = carry[_G:]
            new_vals, new_idxs = [], []
            base = _DTC + g * _C + 2 * d
            d1vec = jnp.full((_L,), base, dtype=jnp.int32)
            d2vec = jnp.full((_L,), base + 1, dtype=jnp.int32)
            for l in range(_G):
                v1 = buf[2 * d, pl.ds(l * _L, _L)]
                v2 = buf[2 * d + 1, pl.ds(l * _L, _L)]
                p2 = v2 < v1
                mv = jnp.minimum(v1, v2)
                mi = jnp.where(p2, d2vec, d1vec)
                q = mv < vals[l]
                new_idxs.append(jnp.where(q, mi, idxs[l]))
                new_vals.append(jnp.minimum(vals[l], mv))
            return tuple(new_vals) + tuple(new_idxs)

        def pair_body(p, carry):
            for j in range(2):
                g = p * 2 + j
                buf, sem = bufs[j], sems[j]
                pltpu.make_async_copy(chunk_src(b, g), buf, sem).wait()
                carry = lax.fori_loop(
                    0, _C // 2, functools.partial(row_body, buf=buf, g=g),
                    carry, unroll=4)

                @pl.when(g + 2 < _NCHUNK)
                def _():
                    pltpu.make_async_copy(chunk_src(b, g + 2), buf, sem).start()
            return carry

        init = tuple(jnp.full((_L,), jnp.inf, dtype=jnp.float32)
                     for _ in range(_G))
        init += tuple(jnp.zeros((_L,), dtype=jnp.int32) for _ in range(_G))
        carry = lax.fori_loop(0, _NCHUNK // 2, pair_body, init)

        for l in range(_G):
            outf[bi, pl.ds(l * _L, _L)] = carry[l]
            outi[bi, pl.ds(l * _L, _L)] = carry[_G + l]

    pltpu.sync_copy(outi, oidx_hbm.at[pl.ds(wid * _BPW, _BPW), :])
    pltpu.sync_copy(outf, oval_hbm.at[pl.ds(wid * _BPW, _BPW), :])


def _sc_argmin(x):
    mesh = plsc.VectorSubcoreMesh(core_axis_name="c", subcore_axis_name="s")
    f = pl.kernel(
        _sc_body,
        out_type=(
            jax.ShapeDtypeStruct((_B, _D2), jnp.int32),
            jax.ShapeDtypeStruct((_B, _D2), jnp.float32),
        ),
        mesh=mesh,
        scratch_types=[
            pltpu.VMEM((_C, _D2), jnp.float32),
            pltpu.VMEM((_C, _D2), jnp.float32),
            pltpu.VMEM((_BPW, _D2), jnp.int32),
            pltpu.VMEM((_BPW, _D2), jnp.float32),
            pltpu.SemaphoreType.DMA,
            pltpu.SemaphoreType.DMA,
        ],
    )
    return f(x)


def _merge_body(tv_ref, ti_ref, sv_ref, si_ref, o_ref):
    sc_wins = sv_ref[...] < tv_ref[...]
    o_ref[...] = jnp.where(sc_wins, si_ref[...], ti_ref[...])


def _merge(tval, tidx, sval, sidx):
    return pl.pallas_call(
        _merge_body,
        out_shape=jax.ShapeDtypeStruct((_B, _D2), jnp.int32),
    )(tval, tidx, sval, sidx)


def kernel(x):
    sidx, sval = _sc_argmin(x)
    tval, tidx = _tc_argmin(x)
    return _merge(tval, tidx, sval, sidx)


# speedup vs baseline: 1.0924x; 1.0017x over previous
import functools

import jax
import jax.numpy as jnp
from jax import lax
from jax.experimental import pallas as pl
from jax.experimental.pallas import tpu as pltpu
from jax.experimental.pallas import tpu_sc as plsc

_B, _D1, _D2 = 64, 8192, 128
_DTC = 4096
_DSC = _D1 - _DTC


_UNROLL = 8


def _tc_body(x_ref, oval_ref, oidx_ref):
    d1 = x_ref.shape[1]
    rows_per_iter = 8 * _UNROLL
    n_iter = d1 // rows_per_iter

    sub_iota = lax.broadcasted_iota(jnp.int32, (8, _D2), 0)

    def step(i, carry):
        rm, ri = carry
        v64 = x_ref[0, pl.ds(i * rows_per_iter, rows_per_iter), :]
        vv = v64.reshape(_UNROLL, 8, _D2)
        base = i * rows_per_iter
        for k in range(_UNROLL):
            v = vv[k]
            lt = v < rm
            ri = jnp.where(lt, sub_iota + (base + k * 8), ri)
            rm = jnp.where(lt, v, rm)
        return rm, ri

    rm0 = jnp.full((8, _D2), jnp.inf, dtype=jnp.float32)
    ri0 = jnp.zeros((8, _D2), dtype=jnp.int32)
    rm, ri = lax.fori_loop(0, n_iter, step, (rm0, ri0))

    m = jnp.min(rm, axis=0)
    idx = jnp.min(jnp.where(rm == m[None, :], ri, 2**30), axis=0)
    oval_ref[0, 0] = m
    oidx_ref[0, 0] = idx


def _tc_argmin(x):
    val3, idx3 = pl.pallas_call(
        _tc_body,
        grid=(_B,),
        in_specs=[pl.BlockSpec((1, _DTC, _D2), lambda i: (i, 0, 0))],
        out_specs=[
            pl.BlockSpec((1, 1, _D2), lambda i: (i, 0, 0)),
            pl.BlockSpec((1, 1, _D2), lambda i: (i, 0, 0)),
        ],
        out_shape=[
            jax.ShapeDtypeStruct((_B, 1, _D2), jnp.float32),
            jax.ShapeDtypeStruct((_B, 1, _D2), jnp.int32),
        ],
    )(x)
    return val3, idx3


_NC, _NS, _L = 2, 16, 16
_NW = _NC * _NS
_BPW = _B // _NW
_C = 256
_NCHUNK = _DSC // _C
_G = _D2 // _L


def _sc_body(x_hbm, oidx_hbm, oval_hbm, buf0, buf1, outi, outf, sem0, sem1):
    wid = lax.axis_index("s") * _NC + lax.axis_index("c")
    bufs = (buf0, buf1)
    sems = (sem0, sem1)

    def chunk_src(b, g):
        return x_hbm.at[b, pl.ds(_DTC + g * _C, _C), :]

    for bi in range(_BPW):
        b = wid * _BPW + bi
        pltpu.make_async_copy(chunk_src(b, 0), buf0, sem0).start()
        pltpu.make_async_copy(chunk_src(b, 1), buf1, sem1).start()

        def row_body(d, carry, buf, g):
            vals = carry[:_G]
            idxs = carry[_G:]
            new_vals, new_idxs = [], []
            base = _DTC + g * _C + 2 * d
            d1vec = jnp.full((_L,), base, dtype=jnp.int32)
            d2vec = jnp.full((_L,), base + 1, dtype=jnp.int32)
            for l in range(_G):
                v1 = buf[2 * d, pl.ds(l * _L, _L)]
                v2 = buf[2 * d + 1, pl.ds(l * _L, _L)]
                p2 = v2 < v1
                mv = jnp.minimum(v1, v2)
                mi = jnp.where(p2, d2vec, d1vec)
                q = mv < vals[l]
                new_idxs.append(jnp.where(q, mi, idxs[l]))
                new_vals.append(jnp.minimum(vals[l], mv))
            return tuple(new_vals) + tuple(new_idxs)

        def pair_body(p, carry):
            for j in range(2):
                g = p * 2 + j
                buf, sem = bufs[j], sems[j]
                pltpu.make_async_copy(chunk_src(b, g), buf, sem).wait()
                carry = lax.fori_loop(
                    0, _C // 2, functools.partial(row_body, buf=buf, g=g),
                    carry, unroll=4)

                @pl.when(g + 2 < _NCHUNK)
                def _():
                    pltpu.make_async_copy(chunk_src(b, g + 2), buf, sem).start()
            return carry

        init = tuple(jnp.full((_L,), jnp.inf, dtype=jnp.float32)
                     for _ in range(_G))
        init += tuple(jnp.zeros((_L,), dtype=jnp.int32) for _ in range(_G))
        carry = lax.fori_loop(0, _NCHUNK // 2, pair_body, init)

        for l in range(_G):
            outf[bi, pl.ds(l * _L, _L)] = carry[l]
            outi[bi, pl.ds(l * _L, _L)] = carry[_G + l]

    pltpu.sync_copy(outi, oidx_hbm.at[pl.ds(wid * _BPW, _BPW), :])
    pltpu.sync_copy(outf, oval_hbm.at[pl.ds(wid * _BPW, _BPW), :])


def _sc_argmin(x):
    mesh = plsc.VectorSubcoreMesh(core_axis_name="c", subcore_axis_name="s")
    f = pl.kernel(
        _sc_body,
        out_type=(
            jax.ShapeDtypeStruct((_B, _D2), jnp.int32),
            jax.ShapeDtypeStruct((_B, _D2), jnp.float32),
        ),
        mesh=mesh,
        scratch_types=[
            pltpu.VMEM((_C, _D2), jnp.float32),
            pltpu.VMEM((_C, _D2), jnp.float32),
            pltpu.VMEM((_BPW, _D2), jnp.int32),
            pltpu.VMEM((_BPW, _D2), jnp.float32),
            pltpu.SemaphoreType.DMA,
            pltpu.SemaphoreType.DMA,
        ],
    )
    return f(x)


def _merge_body(tv_ref, ti_ref, sv_ref, si_ref, o_ref):
    sc_wins = sv_ref[...] < tv_ref[:, 0, :]
    o_ref[...] = jnp.where(sc_wins, si_ref[...], ti_ref[:, 0, :])


def _merge(tval, tidx, sval, sidx):
    return pl.pallas_call(
        _merge_body,
        out_shape=jax.ShapeDtypeStruct((_B, _D2), jnp.int32),
    )(tval, tidx, sval, sidx)


def kernel(x):
    sidx, sval = _sc_argmin(x)
    tval, tidx = _tc_argmin(x)
    return _merge(tval, tidx, sval, sidx)


# speedup vs baseline: 1.1083x; 1.0146x over previous
import functools

import jax
import jax.numpy as jnp
from jax import lax
from jax.experimental import pallas as pl
from jax.experimental.pallas import tpu as pltpu
from jax.experimental.pallas import tpu_sc as plsc

_B, _D1, _D2 = 64, 8192, 128
_DTC = 3840
_DSC = _D1 - _DTC


_UNROLL = 8


def _tc_body(x_ref, oval_ref, oidx_ref):
    d1 = x_ref.shape[1]
    rows_per_iter = 8 * _UNROLL
    n_iter = d1 // rows_per_iter

    sub_iota = lax.broadcasted_iota(jnp.int32, (8, _D2), 0)

    def step(i, carry):
        rm, ri = carry
        v64 = x_ref[0, pl.ds(i * rows_per_iter, rows_per_iter), :]
        vv = v64.reshape(_UNROLL, 8, _D2)
        base = i * rows_per_iter
        for k in range(_UNROLL):
            v = vv[k]
            lt = v < rm
            ri = jnp.where(lt, sub_iota + (base + k * 8), ri)
            rm = jnp.where(lt, v, rm)
        return rm, ri

    rm0 = jnp.full((8, _D2), jnp.inf, dtype=jnp.float32)
    ri0 = jnp.zeros((8, _D2), dtype=jnp.int32)
    rm, ri = lax.fori_loop(0, n_iter, step, (rm0, ri0))

    m = jnp.min(rm, axis=0)
    idx = jnp.min(jnp.where(rm == m[None, :], ri, 2**30), axis=0)
    oval_ref[0, 0] = m
    oidx_ref[0, 0] = idx


def _tc_argmin(x):
    val3, idx3 = pl.pallas_call(
        _tc_body,
        grid=(_B,),
        in_specs=[pl.BlockSpec((1, _DTC, _D2), lambda i: (i, 0, 0))],
        out_specs=[
            pl.BlockSpec((1, 1, _D2), lambda i: (i, 0, 0)),
            pl.BlockSpec((1, 1, _D2), lambda i: (i, 0, 0)),
        ],
        out_shape=[
            jax.ShapeDtypeStruct((_B, 1, _D2), jnp.float32),
            jax.ShapeDtypeStruct((_B, 1, _D2), jnp.int32),
        ],
    )(x)
    return val3, idx3


_NC, _NS, _L = 2, 16, 16
_NW = _NC * _NS
_BPW = _B // _NW
_C = 256
_NCHUNK = _DSC // _C
_G = _D2 // _L


def _sc_body(x_hbm, oidx_hbm, oval_hbm, buf0, buf1, outi, outf, sem0, sem1):
    wid = lax.axis_index("s") * _NC + lax.axis_index("c")
    bufs = (buf0, buf1)
    sems = (sem0, sem1)

    def chunk_src(b, g):
        return x_hbm.at[b, pl.ds(_DTC + g * _C, _C), :]

    for bi in range(_BPW):
        b = wid * _BPW + bi
        pltpu.make_async_copy(chunk_src(b, 0), buf0, sem0).start()
        pltpu.make_async_copy(chunk_src(b, 1), buf1, sem1).start()

        def row_body(d, carry, buf, g):
            vals = carry[:_G]
            idxs = carry[_G:]
            new_vals, new_idxs = [], []
            base = _DTC + g * _C + 2 * d
            d1vec = jnp.full((_L,), base, dtype=jnp.int32)
            d2vec = jnp.full((_L,), base + 1, dtype=jnp.int32)
            for l in range(_G):
                v1 = buf[2 * d, pl.ds(l * _L, _L)]
                v2 = buf[2 * d + 1, pl.ds(l * _L, _L)]
                p2 = v2 < v1
                mv = jnp.minimum(v1, v2)
                mi = jnp.where(p2, d2vec, d1vec)
                q = mv < vals[l]
                new_idxs.append(jnp.where(q, mi, idxs[l]))
                new_vals.append(jnp.minimum(vals[l], mv))
            return tuple(new_vals) + tuple(new_idxs)

        def pair_body(p, carry):
            for j in range(2):
                g = p * 2 + j
                buf, sem = bufs[j], sems[j]
                pltpu.make_async_copy(chunk_src(b, g), buf, sem).wait()
                carry = lax.fori_loop(
                    0, _C // 2, functools.partial(row_body, buf=buf, g=g),
                    carry, unroll=4)

                @pl.when(g + 2 < _NCHUNK)
                def _():
                    pltpu.make_async_copy(chunk_src(b, g + 2), buf, sem).start()
            return carry

        init = tuple(jnp.full((_L,), jnp.inf, dtype=jnp.float32)
                     for _ in range(_G))
        init += tuple(jnp.zeros((_L,), dtype=jnp.int32) for _ in range(_G))
        carry = lax.fori_loop(0, _NCHUNK // 2, pair_body, init)
        if _NCHUNK % 2:
            g = _NCHUNK - 1
            pltpu.make_async_copy(chunk_src(b, g), buf0, sem0).wait()
            carry = lax.fori_loop(
                0, _C // 2, functools.partial(row_body, buf=buf0, g=g),
                carry, unroll=4)

        for l in range(_G):
            outf[bi, pl.ds(l * _L, _L)] = carry[l]
            outi[bi, pl.ds(l * _L, _L)] = carry[_G + l]

    pltpu.sync_copy(outi, oidx_hbm.at[pl.ds(wid * _BPW, _BPW), :])
    pltpu.sync_copy(outf, oval_hbm.at[pl.ds(wid * _BPW, _BPW), :])


def _sc_argmin(x):
    mesh = plsc.VectorSubcoreMesh(core_axis_name="c", subcore_axis_name="s")
    f = pl.kernel(
        _sc_body,
        out_type=(
            jax.ShapeDtypeStruct((_B, _D2), jnp.int32),
            jax.ShapeDtypeStruct((_B, _D2), jnp.float32),
        ),
        mesh=mesh,
        scratch_types=[
            pltpu.VMEM((_C, _D2), jnp.float32),
            pltpu.VMEM((_C, _D2), jnp.float32),
            pltpu.VMEM((_BPW, _D2), jnp.int32),
            pltpu.VMEM((_BPW, _D2), jnp.float32),
            pltpu.SemaphoreType.DMA,
            pltpu.SemaphoreType.DMA,
        ],
    )
    return f(x)


def _merge_body(tv_ref, ti_ref, sv_ref, si_ref, o_ref):
    sc_wins = sv_ref[...] < tv_ref[:, 0, :]
    o_ref[...] = jnp.where(sc_wins, si_ref[...], ti_ref[:, 0, :])


def _merge(tval, tidx, sval, sidx):
    return pl.pallas_call(
        _merge_body,
        out_shape=jax.ShapeDtypeStruct((_B, _D2), jnp.int32),
    )(tval, tidx, sval, sidx)


def kernel(x):
    sidx, sval = _sc_argmin(x)
    tval, tidx = _tc_argmin(x)
    return _merge(tval, tidx, sval, sidx)


# speedup vs baseline: 1.1288x; 1.0185x over previous
import functools

import jax
import jax.numpy as jnp
from jax import lax
from jax.experimental import pallas as pl
from jax.experimental.pallas import tpu as pltpu
from jax.experimental.pallas import tpu_sc as plsc

_B, _D1, _D2 = 64, 8192, 128
_DTC = 3584
_DSC = _D1 - _DTC


_UNROLL = 8


def _tc_body(x_ref, oval_ref, oidx_ref):
    d1 = x_ref.shape[1]
    rows_per_iter = 8 * _UNROLL
    n_iter = d1 // rows_per_iter

    sub_iota = lax.broadcasted_iota(jnp.int32, (8, _D2), 0)

    def step(i, carry):
        rm, ri = carry
        v64 = x_ref[0, pl.ds(i * rows_per_iter, rows_per_iter), :]
        vv = v64.reshape(_UNROLL, 8, _D2)
        base = i * rows_per_iter
        for k in range(_UNROLL):
            v = vv[k]
            lt = v < rm
            ri = jnp.where(lt, sub_iota + (base + k * 8), ri)
            rm = jnp.where(lt, v, rm)
        return rm, ri

    rm0 = jnp.full((8, _D2), jnp.inf, dtype=jnp.float32)
    ri0 = jnp.zeros((8, _D2), dtype=jnp.int32)
    rm, ri = lax.fori_loop(0, n_iter, step, (rm0, ri0))

    m = jnp.min(rm, axis=0)
    idx = jnp.min(jnp.where(rm == m[None, :], ri, 2**30), axis=0)
    oval_ref[0, 0] = m
    oidx_ref[0, 0] = idx


def _tc_argmin(x):
    val3, idx3 = pl.pallas_call(
        _tc_body,
        grid=(_B,),
        in_specs=[pl.BlockSpec((1, _DTC, _D2), lambda i: (i, 0, 0))],
        out_specs=[
            pl.BlockSpec((1, 1, _D2), lambda i: (i, 0, 0)),
            pl.BlockSpec((1, 1, _D2), lambda i: (i, 0, 0)),
        ],
        out_shape=[
            jax.ShapeDtypeStruct((_B, 1, _D2), jnp.float32),
            jax.ShapeDtypeStruct((_B, 1, _D2), jnp.int32),
        ],
    )(x)
    return val3, idx3


_NC, _NS, _L = 2, 16, 16
_NW = _NC * _NS
_BPW = _B // _NW
_C = 256
_NCHUNK = _DSC // _C
_G = _D2 // _L


def _sc_body(x_hbm, oidx_hbm, oval_hbm, buf0, buf1, outi, outf, sem0, sem1):
    wid = lax.axis_index("s") * _NC + lax.axis_index("c")
    bufs = (buf0, buf1)
    sems = (sem0, sem1)

    def chunk_src(b, g):
        return x_hbm.at[b, pl.ds(_DTC + g * _C, _C), :]

    for bi in range(_BPW):
        b = wid * _BPW + bi
        pltpu.make_async_copy(chunk_src(b, 0), buf0, sem0).start()
        pltpu.make_async_copy(chunk_src(b, 1), buf1, sem1).start()

        def row_body(d, carry, buf, g):
            vals = carry[:_G]
            idxs = carry[_G:]
            new_vals, new_idxs = [], []
            base = _DTC + g * _C + 2 * d
            d1vec = jnp.full((_L,), base, dtype=jnp.int32)
            d2vec = jnp.full((_L,), base + 1, dtype=jnp.int32)
            for l in range(_G):
                v1 = buf[2 * d, pl.ds(l * _L, _L)]
                v2 = buf[2 * d + 1, pl.ds(l * _L, _L)]
                p2 = v2 < v1
                mv = jnp.minimum(v1, v2)
                mi = jnp.where(p2, d2vec, d1vec)
                q = mv < vals[l]
                new_idxs.append(jnp.where(q, mi, idxs[l]))
                new_vals.append(jnp.minimum(vals[l], mv))
            return tuple(new_vals) + tuple(new_idxs)

        def pair_body(p, carry):
            for j in range(2):
                g = p * 2 + j
                buf, sem = bufs[j], sems[j]
                pltpu.make_async_copy(chunk_src(b, g), buf, sem).wait()
                carry = lax.fori_loop(
                    0, _C // 2, functools.partial(row_body, buf=buf, g=g),
                    carry, unroll=4)

                @pl.when(g + 2 < _NCHUNK)
                def _():
                    pltpu.make_async_copy(chunk_src(b, g + 2), buf, sem).start()
            return carry

        init = tuple(jnp.full((_L,), jnp.inf, dtype=jnp.float32)
                     for _ in range(_G))
        init += tuple(jnp.zeros((_L,), dtype=jnp.int32) for _ in range(_G))
        carry = lax.fori_loop(0, _NCHUNK // 2, pair_body, init)
        if _NCHUNK % 2:
            g = _NCHUNK - 1
            pltpu.make_async_copy(chunk_src(b, g), buf0, sem0).wait()
            carry = lax.fori_loop(
                0, _C // 2, functools.partial(row_body, buf=buf0, g=g),
                carry, unroll=4)

        for l in range(_G):
            outf[bi, pl.ds(l * _L, _L)] = carry[l]
            outi[bi, pl.ds(l * _L, _L)] = carry[_G + l]

    pltpu.sync_copy(outi, oidx_hbm.at[pl.ds(wid * _BPW, _BPW), :])
    pltpu.sync_copy(outf, oval_hbm.at[pl.ds(wid * _BPW, _BPW), :])


def _sc_argmin(x):
    mesh = plsc.VectorSubcoreMesh(core_axis_name="c", subcore_axis_name="s")
    f = pl.kernel(
        _sc_body,
        out_type=(
            jax.ShapeDtypeStruct((_B, _D2), jnp.int32),
            jax.ShapeDtypeStruct((_B, _D2), jnp.float32),
        ),
        mesh=mesh,
        scratch_types=[
            pltpu.VMEM((_C, _D2), jnp.float32),
            pltpu.VMEM((_C, _D2), jnp.float32),
            pltpu.VMEM((_BPW, _D2), jnp.int32),
            pltpu.VMEM((_BPW, _D2), jnp.float32),
            pltpu.SemaphoreType.DMA,
            pltpu.SemaphoreType.DMA,
        ],
    )
    return f(x)


def _merge_body(tv_ref, ti_ref, sv_ref, si_ref, o_ref):
    sc_wins = sv_ref[...] < tv_ref[:, 0, :]
    o_ref[...] = jnp.where(sc_wins, si_ref[...], ti_ref[:, 0, :])


def _merge(tval, tidx, sval, sidx):
    return pl.pallas_call(
        _merge_body,
        out_shape=jax.ShapeDtypeStruct((_B, _D2), jnp.int32),
    )(tval, tidx, sval, sidx)


def kernel(x):
    sidx, sval = _sc_argmin(x)
    tval, tidx = _tc_argmin(x)
    return _merge(tval, tidx, sval, sidx)
